# baseline (speedup 1.0000x reference)
import functools

import jax
import jax.numpy as jnp
from jax.experimental import pallas as pl
from jax.experimental.pallas import tpu as pltpu

_NH = 8
_NWIN = 7
_TOPK = 4


def _route_kernel(x_ref, wq_ref, wk_ref, bq_ref, bk_ref, idx_ref, *, rs):
    c, hw = x_ref.shape[1], x_ref.shape[2]
    nwin = _NWIN
    nreg = nwin * nwin
    x = x_ref[0]
    dims = (((0,), (0,)), ((), ()))
    qf = jax.lax.dot_general(x, wq_ref[...], dims,
                             preferred_element_type=jnp.float32) + bq_ref[...]
    kf = jax.lax.dot_general(x, wk_ref[...], dims,
                             preferred_element_type=jnp.float32) + bk_ref[...]
    shp5 = (nwin, rs[0], nwin, rs[1], c)
    qp = jnp.mean(qf.reshape(shp5), axis=(1, 3)).reshape(nreg, c)
    kp = jnp.mean(kf.reshape(shp5), axis=(1, 3)).reshape(nreg, c)
    a = jax.lax.dot_general(qp, kp, (((1,), (1,)), ((), ())),
                            preferred_element_type=jnp.float32)
    col = jax.lax.broadcasted_iota(jnp.int32, a.shape, 1)
    picks = []
    for _ in range(_TOPK):
        m = jnp.max(a, axis=1, keepdims=True)
        cand = jnp.where(a == m, col, jnp.int32(2 ** 30))
        j = jnp.min(cand, axis=1, keepdims=True)
        picks.append(j)
        a = jnp.where(col == j, -jnp.inf, a)
    idx_ref[0] = jnp.concatenate(picks, axis=1)


def _fused_kernel(idx_ref, x_ref, wq_ref, wk_ref, wv_ref, bq_ref, bk_ref,
                  bv_ref, wl_ref, bl_ref, wo_ref, bo_ref, o_ref,
                  q_scr, k_scr, v_scr, ao_scr, *, scale, nh, rs):
    b = pl.program_id(0)
    nwin = _NWIN
    c = x_ref.shape[1]
    hd = c // nh
    rsq = rs[0] * rs[1]
    hh = nwin * rs[0]
    ww = nwin * rs[1]
    x = x_ref[0]
    dims = (((0,), (0,)), ((), ()))
    q = jax.lax.dot_general(x, wq_ref[...], dims,
                            preferred_element_type=jnp.float32) + bq_ref[...]
    k = jax.lax.dot_general(x, wk_ref[...], dims,
                            preferred_element_type=jnp.float32) + bk_ref[...]
    v = jax.lax.dot_general(x, wv_ref[...], dims,
                            preferred_element_type=jnp.float32) + bv_ref[...]
    shp5 = (nwin, rs[0], nwin, rs[1], c)
    q_scr[...] = (q * scale).reshape(shp5).astype(jnp.bfloat16)
    k_scr[...] = k.reshape(shp5).astype(jnp.bfloat16)
    v_scr[...] = v.reshape(shp5)

    row = jax.lax.broadcasted_iota(jnp.int32, (nh * rsq, c), 0)
    col = jax.lax.broadcasted_iota(jnp.int32, (nh * rsq, c), 1)
    bd_mask = row // rsq == col // hd
    colh = col[:rsq] // hd
    for n in range(nwin * nwin):
        t, rw = n // nwin, n % nwin
        qn = q_scr[t, :, rw].reshape(rsq, c)
        ks, vs = [], []
        for j in range(_TOPK):
            r = idx_ref[b, n, j]
            r1 = r // nwin
            r2 = r % nwin
            ks.append(k_scr[r1, :, r2].reshape(rsq, c))
            vs.append(v_scr[r1, :, r2].reshape(rsq, c))
        kg = jnp.concatenate(ks, axis=0)
        vg = jnp.concatenate(vs, axis=0).astype(jnp.bfloat16)
        qt = jnp.concatenate([qn] * nh, axis=0)
        qbd = jnp.where(bd_mask, qt, jnp.bfloat16(0.0))
        s = jax.lax.dot_general(qbd, kg, (((1,), (1,)), ((), ())),
                                preferred_element_type=jnp.float32)
        m = jnp.max(s, axis=1, keepdims=True)
        e = jnp.exp(s - m)
        p = (e * (1.0 / jnp.sum(e, axis=1, keepdims=True))).astype(jnp.bfloat16)
        ob = jnp.dot(p, vg, preferred_element_type=jnp.float32)
        acc = jnp.zeros((rsq, c), jnp.float32)
        for h in range(nh):
            acc = acc + jnp.where(colh == h, ob[h * rsq:(h + 1) * rsq], 0.0)
        ao_scr[t, :, rw] = acc.reshape(rs[0], rs[1], c)

    acc = ao_scr[...].reshape(hh, ww, c)
    vimg = v_scr[...].reshape(hh, ww, c)
    wl = wl_ref[...]
    zx = jnp.zeros((hh, 1, c), jnp.float32)
    zy = jnp.zeros((1, ww, c), jnp.float32)
    for kx in range(3):
        if kx == 0:
            vx = jnp.concatenate([zx, vimg[:, :ww - 1, :]], axis=1)
        elif kx == 1:
            vx = vimg
        else:
            vx = jnp.concatenate([vimg[:, 1:, :], zx], axis=1)
        for ky in range(3):
            if ky == 0:
                vsh = jnp.concatenate([zy, vx[:hh - 1]], axis=0)
            elif ky == 1:
                vsh = vx
            else:
                vsh = jnp.concatenate([vx[1:], zy], axis=0)
            acc = acc + vsh * wl[ky * 3 + kx][None, None, :]
    acc = acc + bl_ref[...][None]
    t2 = acc.reshape(hh * ww, c)
    out = jax.lax.dot_general(wo_ref[...], t2, (((1,), (1,)), ((), ())),
                              preferred_element_type=jnp.float32)
    o_ref[0] = out + bo_ref[...]


def kernel(x, Wqkv, bqkv, Wlepe, blepe, Wout, bout):
    B, C, H, W = x.shape
    nh = _NH
    hd = C // nh
    nwin = _NWIN
    rs = (H // nwin, W // nwin)
    nreg = nwin * nwin
    hw = H * W
    scale = hd ** -0.5
    f32 = jnp.float32

    x3 = x.reshape(B, C, hw)

    wq_t = Wqkv[:C].T
    wk_t = Wqkv[C:2 * C].T
    wv_t = Wqkv[2 * C:].T
    bq = bqkv[:C].reshape(1, C)
    bk = bqkv[C:2 * C].reshape(1, C)
    bv = bqkv[2 * C:].reshape(1, C)
    wl9 = Wlepe.reshape(C, 9).T

    xblk = pl.BlockSpec((1, C, hw), lambda b: (b, 0, 0))
    full2 = pl.BlockSpec((C, C), lambda b: (0, 0))
    bias2 = pl.BlockSpec((1, C), lambda b: (0, 0))
    idx = pl.pallas_call(
        functools.partial(_route_kernel, rs=rs),
        grid=(B,),
        in_specs=[xblk, full2, full2, bias2, bias2],
        out_specs=pl.BlockSpec((1, nreg, _TOPK), lambda b: (b, 0, 0)),
        out_shape=jax.ShapeDtypeStruct((B, nreg, _TOPK), jnp.int32),
        compiler_params=pltpu.CompilerParams(
            dimension_semantics=("parallel",)),
    )(x3, wq_t, wk_t, bq, bk)

    xblk2 = pl.BlockSpec((1, C, hw), lambda b, idx_ref: (b, 0, 0))
    full2p = pl.BlockSpec((C, C), lambda b, idx_ref: (0, 0))
    bias2p = pl.BlockSpec((1, C), lambda b, idx_ref: (0, 0))
    shp5 = (nwin, rs[0], nwin, rs[1], C)
    grid_spec = pltpu.PrefetchScalarGridSpec(
        num_scalar_prefetch=1,
        grid=(B,),
        in_specs=[xblk2, full2p, full2p, full2p, bias2p, bias2p, bias2p,
                  pl.BlockSpec((9, C), lambda b, idx_ref: (0, 0)),
                  bias2p, full2p,
                  pl.BlockSpec((C, 1), lambda b, idx_ref: (0, 0))],
        out_specs=pl.BlockSpec((1, C, hw), lambda b, idx_ref: (b, 0, 0)),
        scratch_shapes=[pltpu.VMEM(shp5, jnp.bfloat16),
                        pltpu.VMEM(shp5, jnp.bfloat16),
                        pltpu.VMEM(shp5, f32), pltpu.VMEM(shp5, f32)],
    )
    out_cm = pl.pallas_call(
        functools.partial(_fused_kernel, scale=scale, nh=nh, rs=rs),
        grid_spec=grid_spec,
        out_shape=jax.ShapeDtypeStruct((B, C, hw), f32),
        compiler_params=pltpu.CompilerParams(
            dimension_semantics=("arbitrary",)),
    )(idx, x3, wq_t, wk_t, wv_t, bq, bk, bv, wl9, blepe.reshape(1, C),
      Wout, bout.reshape(C, 1))

    return out_cm.reshape(B, C, H, W)
